# baseline (speedup 1.0000x reference)
import functools

import jax
import jax.numpy as jnp
from jax import lax
from jax.experimental import pallas as pl
from jax.experimental.pallas import tpu as pltpu
from jax.experimental.pallas import tpu_sc as plsc

N = 10000
E = 320000
D = 128

NC = 2
NS = 16
NW = NC * NS

CHUNK = 80
CPT = 125
EPT = CPT * CHUNK
STRIPE = 624
NBUF = 3

_mesh = plsc.VectorSubcoreMesh(core_axis_name="c", subcore_axis_name="s")


@functools.partial(
    pl.kernel,
    out_type=jax.ShapeDtypeStruct((NC, N, D), jnp.float32),
    mesh=_mesh,
    scratch_types=[
        pltpu.VMEM((EPT,), jnp.int32),
        pltpu.VMEM((EPT,), jnp.int32),
        pltpu.VMEM((NBUF, CHUNK, D), jnp.float32),
        pltpu.VMEM_SHARED((N, D), jnp.float32),
        pltpu.SemaphoreType.DMA((NBUF,)),
        pltpu.SemaphoreType.DMA((NBUF,)),
    ],
)
def _sc_segment_sum(x_hbm, src_hbm, dst_hbm, out_hbm,
                    src_v, dst_v, rows_v, acc_s, gsem, ssem):
    cid = lax.axis_index("c")
    sid = lax.axis_index("s")
    wid = cid * NS + sid

    pltpu.sync_copy(src_hbm.at[wid], src_v)
    pltpu.sync_copy(dst_hbm.at[wid], dst_v)

    def _gather_start(j, b):
        pltpu.async_copy(x_hbm.at[src_v.at[pl.ds(j * CHUNK, CHUNK)]],
                         rows_v.at[b], gsem.at[b])

    def _gather_wait(j, b):
        pltpu.make_async_copy(x_hbm.at[src_v.at[pl.ds(j * CHUNK, CHUNK)]],
                              rows_v.at[b], gsem.at[b]).wait()

    def _scatter_start(j, b):
        pltpu.async_copy(rows_v.at[b],
                         acc_s.at[dst_v.at[pl.ds(j * CHUNK, CHUNK)]],
                         ssem.at[b], add=True)

    def _scatter_wait(b):
        pltpu.make_async_copy(rows_v.at[b], acc_s.at[pl.ds(0, CHUNK)],
                              ssem.at[b]).wait()

    def _body(j, b, first=False, last=False):
        br = (b + 2) % NBUF

        _gather_wait(j, b)

        if not first:
            _scatter_wait(br)

        if not last:
            _gather_start(j + 2, br)

        _scatter_start(j, b)

    _gather_start(0, 0)
    _gather_start(1, 1)

    zv = jnp.zeros((16,), jnp.float32)

    @pl.loop(0, CHUNK)
    def _zero_fill(i):
        for k in range(D // 16):
            rows_v[2, i, pl.ds(k * 16, 16)] = zv

    for t in range(STRIPE // CHUNK):
        pltpu.sync_copy(rows_v.at[2],
                        acc_s.at[pl.ds(sid * STRIPE + t * CHUNK, CHUNK)])

    @pl.when(sid == NS - 1)
    def _():
        pltpu.sync_copy(rows_v.at[2], acc_s.at[pl.ds(N - CHUNK, CHUNK)])

    @pl.when(sid != NS - 1)
    def _():
        pltpu.sync_copy(
            rows_v.at[2, pl.ds(0, STRIPE - (STRIPE // CHUNK) * CHUNK)],
            acc_s.at[pl.ds(sid * STRIPE + (STRIPE // CHUNK) * CHUNK,
                           STRIPE - (STRIPE // CHUNK) * CHUNK)])

    plsc.subcore_barrier()

    _body(0, 0, first=True)

    @pl.loop(1, 121, step=NBUF)
    def _edges(jv):
        for u in range(NBUF):
            _body(jv + u, (1 + u) % NBUF)

    for j in range(121, CPT):
        _body(j, j % NBUF, last=(j + 2 >= CPT))
    _scatter_wait((CPT - 1) % NBUF)

    plsc.subcore_barrier()

    pltpu.sync_copy(acc_s.at[pl.ds(sid * STRIPE, STRIPE)],
                    out_hbm.at[cid, pl.ds(sid * STRIPE, STRIPE)])

    @pl.when(sid == NS - 1)
    def _():
        pltpu.sync_copy(acc_s.at[pl.ds(NS * STRIPE, N - NS * STRIPE)],
                        out_hbm.at[cid, pl.ds(NS * STRIPE, N - NS * STRIPE)])


_TC_BLOCK = 2000


def _mlp_body(eps_ref, x_ref, s_ref, w1_ref, b1_ref, w2_ref, b2_ref, o_ref):
    z = (1.0 + eps_ref[0]) * x_ref[...] + s_ref[0] + s_ref[1]
    h = jnp.maximum(
        jnp.dot(z, w1_ref[...], preferred_element_type=jnp.float32) + b1_ref[...],
        0.0)
    o_ref[...] = (
        jnp.dot(h, w2_ref[...], preferred_element_type=jnp.float32) + b2_ref[...])


def _tc_mlp(x, s, eps, w1, b1, w2, b2):
    return pl.pallas_call(
        _mlp_body,
        grid=(N // _TC_BLOCK,),
        in_specs=[
            pl.BlockSpec(memory_space=pltpu.SMEM),
            pl.BlockSpec((_TC_BLOCK, D), lambda i: (i, 0)),
            pl.BlockSpec((NC, _TC_BLOCK, D), lambda i: (0, i, 0)),
            pl.BlockSpec((D, D), lambda i: (0, 0)),
            pl.BlockSpec((1, D), lambda i: (0, 0)),
            pl.BlockSpec((D, D), lambda i: (0, 0)),
            pl.BlockSpec((1, D), lambda i: (0, 0)),
        ],
        out_specs=pl.BlockSpec((_TC_BLOCK, D), lambda i: (i, 0)),
        out_shape=jax.ShapeDtypeStruct((N, D), jnp.float32),
    )(eps, x, s, w1, b1, w2, b2)


def kernel(X, edge_index, eps_0, W1_0, b1_0, W2_0, b2_0,
           eps_1, W1_1, b1_1, W2_1, b2_1,
           eps_2, W1_2, b1_2, W2_2, b2_2):
    src = edge_index[0].reshape(NW, EPT)
    dst = edge_index[1].reshape(NW, EPT)
    params = [
        (eps_0, W1_0, b1_0, W2_0, b2_0),
        (eps_1, W1_1, b1_1, W2_1, b2_1),
        (eps_2, W1_2, b1_2, W2_2, b2_2),
    ]
    x = X
    for (eps, w1, b1, w2, b2) in params:
        s = _sc_segment_sum(x, src, dst)
        x = _tc_mlp(x, s, eps, w1, b1.reshape(1, D), w2, b2.reshape(1, D))
    return x
